# baseline (speedup 1.0000x reference)
import functools

import jax
import jax.numpy as jnp
from jax import lax
from jax.experimental import pallas as pl
from jax.experimental.pallas import tpu as pltpu
from jax.experimental.pallas import tpu_sc as plsc

_D = 2048
_H = 8192
_N = 8192
_K = 6
_BH = 512
_BN = 1024
_L = 16

_NA = _H // _BH
_NB = _N // _BN
_CW = 8


def _ab_body(q_ref, b0_ref, w0_ref, w1_ref, key_ref, idx_ref,
             v_ref, cv_ref, ci_ref):
    g = pl.program_id(0)

    @pl.when(g < _NA)
    def _a():
        qt = (jnp.sum(w0_ref[...] * q_ref[...], axis=1, keepdims=True)
              + b0_ref[...])
        part = jnp.sum(w1_ref[...] * qt, axis=0, keepdims=True)

        @pl.when(g == 0)
        def _init():
            v_ref[...] = part

        @pl.when(g > 0)
        def _acc():
            v_ref[...] += part

    @pl.when(g >= _NA)
    def _b():
        b = g - _NA
        v3 = v_ref[...].reshape(1, 1, _D)
        scores = jnp.sum(key_ref[...] * v3, axis=2)
        lin = (b * _BN
               + lax.broadcasted_iota(jnp.int32, (_BN // 128, 128), 0) * 128
               + lax.broadcasted_iota(jnp.int32, (_BN // 128, 128), 1))
        slots = lax.broadcasted_iota(jnp.int32, (1, _CW), 1)
        rowv = jnp.full((1, _CW), -jnp.inf, jnp.float32)
        rowi = jnp.full((1, _CW), 2**30, jnp.int32)
        for t in range(_K):
            mx = jnp.max(scores)
            am = jnp.min(jnp.where(scores == mx, lin, jnp.int32(2**30)))
            scores = jnp.where(lin == am, jnp.float32(-100000.0), scores)
            rowv = jnp.where(slots == t, mx, rowv)
            rowi = jnp.where(slots == t, am, rowi)
        cv_ref[pl.ds(b, 1), :] = rowv
        ci_ref[pl.ds(b, 1), :] = rowi

        @pl.when(g == _NA + _NB - 1)
        def _merge():
            vals = cv_ref[...]
            idxs = ci_ref[...]
            lanes = lax.broadcasted_iota(jnp.int32, (1, _L), 1)
            out = jnp.zeros((1, _L), jnp.int32)
            for t in range(_K):
                mx = jnp.max(vals)
                gi = jnp.min(jnp.where(vals == mx, idxs, jnp.int32(2**30)))
                vals = jnp.where(idxs == gi, jnp.float32(-jnp.inf), vals)
                out = jnp.where(lanes == t, gi, out)
            idx_ref[...] = out


def _gather_mean_body(idx_hbm, key_hbm, out_hbm, idx_v, rows_v, out_v, sem):
    cid = lax.axis_index("c")
    sid = lax.axis_index("s")

    @pl.when(jnp.logical_and(cid == 0, sid == 0))
    def _():
        pltpu.sync_copy(idx_hbm, idx_v)
        pltpu.async_copy(key_hbm.at[idx_v], rows_v, sem).wait()

        def mean_chunk(d, _):
            acc = rows_v[0, pl.ds(d * _L, _L)]
            for j in range(1, _K):
                acc = acc + rows_v[j, pl.ds(d * _L, _L)]
            out_v[pl.ds(d * _L, _L)] = acc * jnp.float32(1.0 / _K)
            return 0

        lax.fori_loop(0, _D // _L, mean_chunk, 0)
        pltpu.sync_copy(out_v, out_hbm)


@functools.cache
def _gather_mean():
    return pl.kernel(
        _gather_mean_body,
        out_type=jax.ShapeDtypeStruct((_D,), jnp.float32),
        mesh=plsc.VectorSubcoreMesh(core_axis_name="c", subcore_axis_name="s",
                                    num_cores=1),
        scratch_types=[
            pltpu.VMEM((_L,), jnp.int32),
            pltpu.VMEM((_L, _D), jnp.float32),
            pltpu.VMEM((_D,), jnp.float32),
            pltpu.SemaphoreType.DMA,
        ],
    )


def kernel(query, key, W0, b0, W1, b1):
    del b1
    qrow = query.reshape(1, _D)
    b0col = b0.reshape(_H, 1)

    idx = pl.pallas_call(
        _ab_body,
        grid=(_NA + _NB,),
        in_specs=[
            pl.BlockSpec((1, _D), lambda g: (0, 0)),
            pl.BlockSpec((_BH, 1), lambda g: (jnp.minimum(g, _NA - 1), 0)),
            pl.BlockSpec((_BH, _D), lambda g: (jnp.minimum(g, _NA - 1), 0)),
            pl.BlockSpec((_BH, _D), lambda g: (jnp.minimum(g, _NA - 1), 0)),
            pl.BlockSpec((_BN // 128, 128, _D),
                         lambda g: (jnp.maximum(g - _NA, 0), 0, 0)),
        ],
        out_specs=pl.BlockSpec((1, _L), lambda g: (0, 0)),
        out_shape=jax.ShapeDtypeStruct((1, _L), jnp.int32),
        scratch_shapes=[
            pltpu.VMEM((1, _D), jnp.float32),
            pltpu.VMEM((_NB, _CW), jnp.float32),
            pltpu.VMEM((_NB, _CW), jnp.int32),
        ],
    )(qrow, b0col, W0, W1, key.reshape(_N // 128, 128, _D))

    return _gather_mean()(idx.reshape(_L), key)


# speedup vs baseline: 1.0164x; 1.0164x over previous
import functools

import jax
import jax.numpy as jnp
from jax import lax
from jax.experimental import pallas as pl
from jax.experimental.pallas import tpu as pltpu
from jax.experimental.pallas import tpu_sc as plsc

_D = 2048
_H = 8192
_N = 8192
_K = 6
_BH = 512
_BN = 1024
_L = 16

_NA = _H // _BH
_NB = _N // _BN
_CW = 8


def _ab_body(q_ref, b0_ref, w0_ref, w1_ref, key_ref, idx_ref,
             v_ref, cv_ref, ci_ref):
    g = pl.program_id(0)

    @pl.when(g < _NA)
    def _a():
        qt = (jnp.sum(w0_ref[...] * q_ref[...], axis=1, keepdims=True)
              + b0_ref[...])
        part = jnp.sum(w1_ref[...] * qt, axis=0, keepdims=True)

        @pl.when(g == 0)
        def _init():
            v_ref[...] = part

        @pl.when(g > 0)
        def _acc():
            v_ref[...] += part

    @pl.when(g >= _NA)
    def _b():
        b = g - _NA
        scores = jnp.sum(key_ref[...] * v_ref[...], axis=1, keepdims=True)
        col = lax.broadcasted_iota(jnp.int32, (_BN, 1), 0)
        slots = lax.broadcasted_iota(jnp.int32, (1, _CW), 1)
        rowv = jnp.full((1, _CW), -jnp.inf, jnp.float32)
        rowi = jnp.full((1, _CW), 2**30, jnp.int32)
        for t in range(_K):
            mx = jnp.max(scores)
            am = jnp.min(jnp.where(scores == mx, col, jnp.int32(2**30)))
            scores = jnp.where(col == am, jnp.float32(-100000.0), scores)
            rowv = jnp.where(slots == t, mx, rowv)
            rowi = jnp.where(slots == t, am + b * _BN, rowi)
        cv_ref[pl.ds(b, 1), :] = rowv
        ci_ref[pl.ds(b, 1), :] = rowi

        @pl.when(g == _NA + _NB - 1)
        def _merge():
            vals = cv_ref[...]
            idxs = ci_ref[...]
            lanes = lax.broadcasted_iota(jnp.int32, (1, _L), 1)
            out = jnp.zeros((1, _L), jnp.int32)
            for t in range(_K):
                mx = jnp.max(vals)
                gi = jnp.min(jnp.where(vals == mx, idxs, jnp.int32(2**30)))
                vals = jnp.where(idxs == gi, jnp.float32(-jnp.inf), vals)
                out = jnp.where(lanes == t, gi, out)
            idx_ref[...] = out


def _gather_mean_body(idx_hbm, key_hbm, out_hbm, idx_v, rows_v, out_v, sem):
    cid = lax.axis_index("c")
    sid = lax.axis_index("s")

    @pl.when(jnp.logical_and(cid == 0, sid == 0))
    def _():
        pltpu.sync_copy(idx_hbm, idx_v)
        pltpu.async_copy(key_hbm.at[idx_v], rows_v, sem).wait()

        def mean_chunk(d, _):
            acc = rows_v[0, pl.ds(d * _L, _L)]
            for j in range(1, _K):
                acc = acc + rows_v[j, pl.ds(d * _L, _L)]
            out_v[pl.ds(d * _L, _L)] = acc * jnp.float32(1.0 / _K)
            return 0

        lax.fori_loop(0, _D // _L, mean_chunk, 0)
        pltpu.sync_copy(out_v, out_hbm)


@functools.cache
def _gather_mean():
    return pl.kernel(
        _gather_mean_body,
        out_type=jax.ShapeDtypeStruct((_D,), jnp.float32),
        mesh=plsc.VectorSubcoreMesh(core_axis_name="c", subcore_axis_name="s",
                                    num_cores=1),
        scratch_types=[
            pltpu.VMEM((_L,), jnp.int32),
            pltpu.VMEM((_L, _D), jnp.float32),
            pltpu.VMEM((_D,), jnp.float32),
            pltpu.SemaphoreType.DMA,
        ],
    )


def kernel(query, key, W0, b0, W1, b1):
    del b1
    qrow = query.reshape(1, _D)
    b0col = b0.reshape(_H, 1)

    idx = pl.pallas_call(
        _ab_body,
        grid=(_NA + _NB,),
        in_specs=[
            pl.BlockSpec((1, _D), lambda g: (0, 0)),
            pl.BlockSpec((_BH, 1), lambda g: (jnp.minimum(g, _NA - 1), 0)),
            pl.BlockSpec((_BH, _D), lambda g: (jnp.minimum(g, _NA - 1), 0)),
            pl.BlockSpec((_BH, _D), lambda g: (jnp.minimum(g, _NA - 1), 0)),
            pl.BlockSpec((_BN, _D), lambda g: (jnp.maximum(g - _NA, 0), 0)),
        ],
        out_specs=pl.BlockSpec((1, _L), lambda g: (0, 0)),
        out_shape=jax.ShapeDtypeStruct((1, _L), jnp.int32),
        scratch_shapes=[
            pltpu.VMEM((1, _D), jnp.float32),
            pltpu.VMEM((_NB, _CW), jnp.float32),
            pltpu.VMEM((_NB, _CW), jnp.int32),
        ],
    )(qrow, b0col, W0, W1, key)

    return _gather_mean()(idx.reshape(_L), key)
